# baseline (speedup 1.0000x reference)
import functools

import jax
import jax.numpy as jnp
from jax.experimental import pallas as pl
from jax.experimental.pallas import tpu as pltpu


def _lora_block(x_ref, tid_ref, a_ref, b_ref, out_ref, a_flat, *, rank):
    n_tasks = a_ref.shape[0]
    n_cols = n_tasks * rank

    @pl.when(pl.program_id(0) == 0)
    def _():
        a_flat[...] = jnp.concatenate([a_ref[e] for e in range(n_tasks)], axis=1)

    tid = jnp.reshape(tid_ref[...], (tid_ref.shape[0], 1))
    col_expert = jax.lax.broadcasted_iota(jnp.int32, (tid.shape[0], n_cols), 1) // rank
    xa = jnp.dot(x_ref[...], a_flat[...], preferred_element_type=jnp.float32)
    xa = jnp.where(tid == col_expert, xa, 0.0)
    out_ref[...] = jnp.dot(xa, b_ref[...], preferred_element_type=jnp.float32)


def kernel(x, task_id, lora_A, lora_B):
    T, in_dim = x.shape
    n_tasks, _, rank = lora_A.shape
    out_dim = lora_B.shape[2]
    er = n_tasks * rank

    b_flat = lora_B.reshape(er, out_dim)

    TB = 1024
    grid = (T // TB,)

    body = functools.partial(_lora_block, rank=rank)
    return pl.pallas_call(
        body,
        grid=grid,
        in_specs=[
            pl.BlockSpec((TB, in_dim), lambda i: (i, 0)),
            pl.BlockSpec((TB,), lambda i: (i,)),
            pl.BlockSpec((n_tasks, in_dim, rank), lambda i: (0, 0, 0)),
            pl.BlockSpec((er, out_dim), lambda i: (0, 0)),
        ],
        out_specs=pl.BlockSpec((TB, out_dim), lambda i: (i, 0)),
        out_shape=jax.ShapeDtypeStruct((T, out_dim), jnp.float32),
        scratch_shapes=[pltpu.VMEM((in_dim, er), jnp.float32)],
    )(x, task_id, lora_A, b_flat)


# speedup vs baseline: 1.4867x; 1.4867x over previous
import functools

import jax
import jax.numpy as jnp
from jax.experimental import pallas as pl


def _lora_block(x_ref, tid_ref, a_ref, b_ref, out_ref, *, rank):
    n_cols = a_ref.shape[1]
    tb = x_ref.shape[0]
    tid = jnp.reshape(tid_ref[...], (tb, 1))
    col_expert = jax.lax.broadcasted_iota(jnp.int32, (tb, n_cols), 1) // rank
    xa = jnp.dot(x_ref[...], a_ref[...], preferred_element_type=jnp.float32)
    xa = jnp.where(tid == col_expert, xa, 0.0)
    out_ref[...] = jnp.dot(xa, b_ref[...], preferred_element_type=jnp.float32)


def kernel(x, task_id, lora_A, lora_B):
    T, in_dim = x.shape
    n_tasks, _, rank = lora_A.shape
    out_dim = lora_B.shape[2]
    er = n_tasks * rank

    a_flat = jnp.transpose(lora_A, (1, 0, 2)).reshape(in_dim, er)
    b_flat = lora_B.reshape(er, out_dim)

    TB = 1024
    grid = (T // TB,)

    body = functools.partial(_lora_block, rank=rank)
    return pl.pallas_call(
        body,
        grid=grid,
        in_specs=[
            pl.BlockSpec((TB, in_dim), lambda i: (i, 0)),
            pl.BlockSpec((TB,), lambda i: (i,)),
            pl.BlockSpec((in_dim, er), lambda i: (0, 0)),
            pl.BlockSpec((er, out_dim), lambda i: (0, 0)),
        ],
        out_specs=pl.BlockSpec((TB, out_dim), lambda i: (i, 0)),
        out_shape=jax.ShapeDtypeStruct((T, out_dim), jnp.float32),
    )(x, task_id, a_flat, b_flat)
